# baseline (speedup 1.0000x reference)
import functools

import jax
import jax.numpy as jnp
import numpy as np
from jax import lax
from jax.experimental import pallas as pl
from jax.experimental.pallas import tpu as pltpu
from jax.experimental.pallas import tpu_sc as plsc

NCLS = 21
B = 4
H = 512
W = 512
HW = H * W
NW = 32
ROWS_PER_W = H // NW
RCHUNK = 4
NCHUNK = ROWS_PER_W // RCHUNK
ACC_PAD = 16 * NCLS + 16

LN2 = 0.6931471805599453


def _ln_tables():
    k = np.arange(256)
    mk = 1.0 + k / 256.0
    mk1 = 1.0 + (k + 1) / 256.0
    bs = (np.log(mk1) - np.log(mk)) * 256.0
    as_ = np.log(mk) - bs * mk
    nb = (-bs).astype(np.float32)
    na2 = (-as_ + 127.0 * LN2).astype(np.float32)
    return na2, nb


_NA2_TAB, _NB_TAB = _ln_tables()


def _focal_main_body(in_hbm, t_hbm, na_hbm, nb_hbm, s_out, f_out, p_buf,
                     t_all, tsum_v, na_v, nb_v, s_acc, f_acc, svec, fvec,
                     sem0, sem1):
    wid = lax.axis_index("s") * 2 + lax.axis_index("c")
    h0 = wid * ROWS_PER_W
    iota = lax.iota(jnp.int32, 16)
    zeros = jnp.zeros((16,), jnp.float32)
    ones = jnp.ones((16,), jnp.float32)
    lane_base = iota * NCLS
    sems = [sem0, sem1]

    def start_chunk(ch, b, slot):
        pltpu.async_copy(
            in_hbm.at[b, :, pl.ds(h0 + ch * RCHUNK, RCHUNK), :],
            p_buf.at[slot],
            sems[slot],
        )

    def wait_chunk(slot):
        pltpu.make_async_copy(
            in_hbm.at[0, :, pl.ds(0, RCHUNK), :], p_buf.at[slot], sems[slot]
        ).wait()

    start_chunk(0, 0, 0)
    pltpu.sync_copy(t_hbm.at[:, :, pl.ds(h0, ROWS_PER_W), :], t_all)
    pltpu.sync_copy(na_hbm, na_v)
    pltpu.sync_copy(nb_hbm, nb_v)

    for k in range(ACC_PAD // 16):
        s_acc[pl.ds(k * 16, 16)] = zeros
        f_acc[pl.ds(k * 16, 16)] = zeros
    for k in range(128 // 16):
        svec[pl.ds(k * 16, 16)] = zeros
        fvec[pl.ds(k * 16, 16)] = zeros

    def g_pass(ch, b, slot):
        for r in range(RCHUNK):
            rr = ch * RCHUNK + r
            rvec = jnp.full((16,), r, jnp.int32)

            def px_body(jj, c2):
                off = jj * 16
                tb = t_all[b, 0, rr, pl.ds(off, 16)]
                pb = plsc.load_gather(
                    p_buf.at[slot], [tb, rvec, off + iota]
                )
                pb = jnp.maximum(pb, 1e-5)
                i = plsc.bitcast(pb, jnp.int32)
                kk = (i >> 15) & 0xFF
                m = plsc.bitcast(
                    (i & 0x007FFFFF) | 0x3F800000, jnp.float32
                )
                na = plsc.load_gather(na_v, [kk])
                nb = plsc.load_gather(nb_v, [kk])
                nlnp = na + nb * m - (i >> 23).astype(jnp.float32) * LN2
                omp = 1.0 - pb
                g = omp * omp * nlnp
                if b == 0:
                    tsum_v[r, pl.ds(off, 16)] = g
                else:
                    plsc.addupdate(tsum_v.at[r, pl.ds(off, 16)], g)
                return c2

            lax.fori_loop(0, W // 16, px_body, 0)

    def scatter_pass(ch):
        for r in range(RCHUNK):
            rr = ch * RCHUNK + r

            def px_body(jj, c2):
                off = jj * 16
                tsv = tsum_v[r, pl.ds(off, 16)]
                for b in range(B):
                    tb = t_all[b, 0, rr, pl.ds(off, 16)]
                    idx = lane_base + tb
                    plsc.addupdate_scatter(s_acc, [idx], tsv)
                    plsc.addupdate_scatter(f_acc, [idx], ones)
                return c2

            lax.fori_loop(0, W // 16, px_body, 0)

    def chunk_body(ch, carry):
        start_chunk(ch, 1, 1)
        wait_chunk(0)
        g_pass(ch, 0, 0)
        start_chunk(ch, 2, 0)
        wait_chunk(1)
        g_pass(ch, 1, 1)
        start_chunk(ch, 3, 1)
        wait_chunk(0)
        g_pass(ch, 2, 0)

        @pl.when(ch + 1 < NCHUNK)
        def _():
            start_chunk(ch + 1, 0, 0)

        wait_chunk(1)
        g_pass(ch, 3, 1)
        scatter_pass(ch)
        return carry

    lax.fori_loop(0, NCHUNK, chunk_body, 0)

    acc_s0 = zeros
    acc_s1 = zeros
    acc_f0 = zeros
    acc_f1 = zeros
    for l in range(16):
        acc_s0 = acc_s0 + s_acc[pl.ds(l * NCLS, 16)]
        acc_s1 = acc_s1 + s_acc[pl.ds(l * NCLS + 5, 16)]
        acc_f0 = acc_f0 + f_acc[pl.ds(l * NCLS, 16)]
        acc_f1 = acc_f1 + f_acc[pl.ds(l * NCLS + 5, 16)]
    svec[pl.ds(5, 16)] = acc_s1
    svec[pl.ds(0, 16)] = acc_s0
    fvec[pl.ds(5, 16)] = acc_f1
    fvec[pl.ds(0, 16)] = acc_f0

    pltpu.sync_copy(svec, s_out.at[wid])
    pltpu.sync_copy(fvec, f_out.at[wid])


_focal_main = functools.partial(
    pl.kernel,
    out_type=[
        jax.ShapeDtypeStruct((NW, 128), jnp.float32),
        jax.ShapeDtypeStruct((NW, 128), jnp.float32),
    ],
    mesh=plsc.VectorSubcoreMesh(core_axis_name="c", subcore_axis_name="s"),
    scratch_types=[
        pltpu.VMEM((2, NCLS, RCHUNK, W), jnp.float32),
        pltpu.VMEM((B, 1, ROWS_PER_W, W), jnp.int32),
        pltpu.VMEM((RCHUNK, W), jnp.float32),
        pltpu.VMEM((256,), jnp.float32),
        pltpu.VMEM((256,), jnp.float32),
        pltpu.VMEM((ACC_PAD,), jnp.float32),
        pltpu.VMEM((ACC_PAD,), jnp.float32),
        pltpu.VMEM((128,), jnp.float32),
        pltpu.VMEM((128,), jnp.float32),
        pltpu.SemaphoreType.DMA,
        pltpu.SemaphoreType.DMA,
    ],
    compiler_params=pltpu.CompilerParams(needs_layout_passes=False),
)(_focal_main_body)


def _combine_body(s_ref, f_ref, o_ref):
    s = jnp.sum(s_ref[...], axis=0)
    f = jnp.sum(f_ref[...], axis=0)
    cw = 1.0 / jnp.log(1.1 + f * (1.0 / float(B * HW)))
    o_ref[...] = jnp.sum(cw * s).reshape(1, 1)


def kernel(input, target):
    t = target.astype(jnp.int32)
    s_tab, f_tab = _focal_main(
        input, t, jnp.asarray(_NA2_TAB), jnp.asarray(_NB_TAB)
    )
    out = pl.pallas_call(
        _combine_body,
        out_shape=jax.ShapeDtypeStruct((1, 1), jnp.float32),
    )(s_tab, f_tab)
    return out[0, 0]


# speedup vs baseline: 1.5492x; 1.5492x over previous
import functools

import jax
import jax.numpy as jnp
import numpy as np
from jax import lax
from jax.experimental import pallas as pl
from jax.experimental.pallas import tpu as pltpu
from jax.experimental.pallas import tpu_sc as plsc

NCLS = 21
B = 4
H = 512
W = 512
HW = H * W
NW = 32
ROWS_PER_W = H // NW
ACC_PAD = 16 * NCLS + 16

LN2 = 0.6931471805599453


def _ln_tables():
    k = np.arange(256)
    mk = 1.0 + k / 256.0
    mk1 = 1.0 + (k + 1) / 256.0
    bs = (np.log(mk1) - np.log(mk)) * 256.0
    as_ = np.log(mk) - bs * mk
    nb = (-bs).astype(np.float32)
    na2 = (-as_ + 127.0 * LN2).astype(np.float32)
    return na2, nb


_NA2_TAB, _NB_TAB = _ln_tables()


def _focal_main_body(in_hbm, t_hbm, na_hbm, nb_hbm, s_out, f_out, in_v, t_v,
                     na_v, nb_v, s_acc, f_acc, svec, fvec, sem0, sem1):
    wid = lax.axis_index("s") * 2 + lax.axis_index("c")
    h0 = wid * ROWS_PER_W
    iota = lax.iota(jnp.int32, 16)
    zeros = jnp.zeros((16,), jnp.float32)
    zeros_i = jnp.zeros((16,), jnp.int32)
    ones = jnp.ones((16,), jnp.float32)
    lane_base = iota * NCLS
    sems = [sem0, sem1]

    pltpu.sync_copy(na_hbm, na_v)
    pltpu.sync_copy(nb_hbm, nb_v)
    for k in range(ACC_PAD // 16):
        s_acc[pl.ds(k * 16, 16)] = zeros
        f_acc[pl.ds(k * 16, 16)] = zeros
    for k in range(128 // 16):
        svec[pl.ds(k * 16, 16)] = zeros
        fvec[pl.ds(k * 16, 16)] = zeros

    CSPLIT = (0, 7, 14, NCLS)

    def start_row(r, slot):
        h = h0 + r
        for b in range(B):
            for c0, c1 in zip(CSPLIT[:-1], CSPLIT[1:]):
                pltpu.async_copy(
                    in_hbm.at[b, pl.ds(c0, c1 - c0), pl.ds(h, 1), :],
                    in_v.at[slot, pl.ds(b * NCLS + c0, c1 - c0)],
                    sems[slot],
                )
        pltpu.async_copy(
            t_hbm.at[:, :, pl.ds(h, 1), :], t_v.at[slot], sems[slot]
        )

    def wait_row(slot):
        for b in range(B):
            for c0, c1 in zip(CSPLIT[:-1], CSPLIT[1:]):
                pltpu.make_async_copy(
                    in_hbm.at[b, pl.ds(c0, c1 - c0), pl.ds(0, 1), :],
                    in_v.at[slot, pl.ds(b * NCLS + c0, c1 - c0)],
                    sems[slot],
                ).wait()
        pltpu.make_async_copy(
            t_hbm.at[:, :, pl.ds(0, 1), :], t_v.at[slot], sems[slot]
        ).wait()

    def do_vreg(slot, off):
        wvec = off + iota
        tsum = jnp.zeros((16,), jnp.float32)
        ts = []
        for b in range(B):
            tb = t_v[slot, b, 0, 0, pl.ds(off, 16)]
            ct = tb + (b * NCLS)
            pb = plsc.load_gather(in_v.at[slot], [ct, zeros_i, wvec])
            pb = jnp.maximum(pb, 1e-5)
            i = plsc.bitcast(pb, jnp.int32)
            kk = (i >> 15) & 0xFF
            m = plsc.bitcast((i & 0x007FFFFF) | 0x3F800000, jnp.float32)
            na = plsc.load_gather(na_v, [kk])
            nb = plsc.load_gather(nb_v, [kk])
            nlnp = na + nb * m - (i >> 23).astype(jnp.float32) * LN2
            omp = 1.0 - pb
            tsum = tsum + omp * omp * nlnp
            ts.append(tb)
        for b in range(B):
            idx = lane_base + ts[b]
            plsc.addupdate_scatter(s_acc, [idx], tsum)
            plsc.addupdate_scatter(f_acc, [idx], ones)

    def compute_row(slot):
        def px_body(jj, c2):
            off = jj * 32
            do_vreg(slot, off)
            do_vreg(slot, off + 16)
            return c2

        lax.fori_loop(0, W // 32, px_body, 0)

    start_row(0, 0)

    def pair_body(pr, carry):
        r0 = pr * 2
        start_row(r0 + 1, 1)
        wait_row(0)
        compute_row(0)

        @pl.when(r0 + 2 < ROWS_PER_W)
        def _():
            start_row(r0 + 2, 0)

        wait_row(1)
        compute_row(1)
        return carry

    lax.fori_loop(0, ROWS_PER_W // 2, pair_body, 0)

    acc_s0 = zeros
    acc_s1 = zeros
    acc_f0 = zeros
    acc_f1 = zeros
    for l in range(16):
        acc_s0 = acc_s0 + s_acc[pl.ds(l * NCLS, 16)]
        acc_s1 = acc_s1 + s_acc[pl.ds(l * NCLS + 5, 16)]
        acc_f0 = acc_f0 + f_acc[pl.ds(l * NCLS, 16)]
        acc_f1 = acc_f1 + f_acc[pl.ds(l * NCLS + 5, 16)]
    svec[pl.ds(5, 16)] = acc_s1
    svec[pl.ds(0, 16)] = acc_s0
    fvec[pl.ds(5, 16)] = acc_f1
    fvec[pl.ds(0, 16)] = acc_f0

    pltpu.sync_copy(svec, s_out.at[wid])
    pltpu.sync_copy(fvec, f_out.at[wid])


_focal_main = functools.partial(
    pl.kernel,
    out_type=[
        jax.ShapeDtypeStruct((NW, 128), jnp.float32),
        jax.ShapeDtypeStruct((NW, 128), jnp.float32),
    ],
    mesh=plsc.VectorSubcoreMesh(core_axis_name="c", subcore_axis_name="s"),
    scratch_types=[
        pltpu.VMEM((2, B * NCLS, 1, W), jnp.float32),
        pltpu.VMEM((2, B, 1, 1, W), jnp.int32),
        pltpu.VMEM((256,), jnp.float32),
        pltpu.VMEM((256,), jnp.float32),
        pltpu.VMEM((ACC_PAD,), jnp.float32),
        pltpu.VMEM((ACC_PAD,), jnp.float32),
        pltpu.VMEM((128,), jnp.float32),
        pltpu.VMEM((128,), jnp.float32),
        pltpu.SemaphoreType.DMA,
        pltpu.SemaphoreType.DMA,
    ],
    compiler_params=pltpu.CompilerParams(needs_layout_passes=False),
)(_focal_main_body)


def _combine_body(s_ref, f_ref, o_ref):
    s = jnp.sum(s_ref[...], axis=0)
    f = jnp.sum(f_ref[...], axis=0)
    cw = 1.0 / jnp.log(1.1 + f * (1.0 / float(B * HW)))
    o_ref[...] = jnp.sum(cw * s).reshape(1, 1)


def kernel(input, target):
    t = target.astype(jnp.int32)
    s_tab, f_tab = _focal_main(
        input, t, jnp.asarray(_NA2_TAB), jnp.asarray(_NB_TAB)
    )
    out = pl.pallas_call(
        _combine_body,
        out_shape=jax.ShapeDtypeStruct((1, 1), jnp.float32),
    )(s_tab, f_tab)
    return out[0, 0]


# speedup vs baseline: 1.5856x; 1.0235x over previous
import functools

import jax
import jax.numpy as jnp
import numpy as np
from jax import lax
from jax.experimental import pallas as pl
from jax.experimental.pallas import tpu as pltpu
from jax.experimental.pallas import tpu_sc as plsc

NCLS = 21
B = 4
H = 512
W = 512
HW = H * W
NW = 32
ROWS_PER_W = H // NW
ACC_PAD = 16 * NCLS + 16

LN2 = 0.6931471805599453


def _ln_tables():
    k = np.arange(256)
    mk = 1.0 + k / 256.0
    mk1 = 1.0 + (k + 1) / 256.0
    bs = (np.log(mk1) - np.log(mk)) * 256.0
    as_ = np.log(mk) - bs * mk
    nb = (-bs).astype(np.float32)
    na2 = (-as_ + 127.0 * LN2).astype(np.float32)
    return na2, nb


_NA2_TAB, _NB_TAB = _ln_tables()


def _focal_main_body(in_hbm, t_hbm, na_hbm, nb_hbm, s_out, f_out, in_v, t_v,
                     na_v, nb_v, s_acc, f_acc, svec, fvec, sem0, sem1):
    wid = lax.axis_index("s") * 2 + lax.axis_index("c")
    h0 = wid * ROWS_PER_W
    iota = lax.iota(jnp.int32, 16)
    zeros = jnp.zeros((16,), jnp.float32)
    zeros_i = jnp.zeros((16,), jnp.int32)
    ones = jnp.ones((16,), jnp.float32)
    lane_base = iota * NCLS
    sems = [sem0, sem1]

    pltpu.sync_copy(na_hbm, na_v)
    pltpu.sync_copy(nb_hbm, nb_v)
    for k in range(ACC_PAD // 16):
        s_acc[pl.ds(k * 16, 16)] = zeros
        f_acc[pl.ds(k * 16, 16)] = zeros
    for k in range(128 // 16):
        svec[pl.ds(k * 16, 16)] = zeros
        fvec[pl.ds(k * 16, 16)] = zeros

    def start_row(r, slot):
        h = h0 + r
        for b in range(B):
            pltpu.async_copy(
                in_hbm.at[b, :, pl.ds(h, 1), :],
                in_v.at[slot, pl.ds(b * NCLS, NCLS)],
                sems[slot],
            )
        pltpu.async_copy(
            t_hbm.at[:, :, pl.ds(h, 1), :], t_v.at[slot], sems[slot]
        )

    def wait_row(slot):
        for b in range(B):
            pltpu.make_async_copy(
                in_hbm.at[b, :, pl.ds(0, 1), :],
                in_v.at[slot, pl.ds(b * NCLS, NCLS)],
                sems[slot],
            ).wait()
        pltpu.make_async_copy(
            t_hbm.at[:, :, pl.ds(0, 1), :], t_v.at[slot], sems[slot]
        ).wait()

    def do_vreg(slot, off):
        wvec = off + iota
        tsum = jnp.zeros((16,), jnp.float32)
        ts = []
        for b in range(B):
            tb = t_v[slot, b, 0, 0, pl.ds(off, 16)]
            ct = tb + (b * NCLS)
            pb = plsc.load_gather(in_v.at[slot], [ct, zeros_i, wvec])
            pb = jnp.maximum(pb, 1e-5)
            i = plsc.bitcast(pb, jnp.int32)
            kk = (i >> 15) & 0xFF
            m = plsc.bitcast((i & 0x007FFFFF) | 0x3F800000, jnp.float32)
            na = plsc.load_gather(na_v, [kk])
            nb = plsc.load_gather(nb_v, [kk])
            nlnp = na + nb * m - (i >> 23).astype(jnp.float32) * LN2
            omp = 1.0 - pb
            tsum = tsum + omp * omp * nlnp
            ts.append(tb)
        for b in range(B):
            idx = lane_base + ts[b]
            plsc.addupdate_scatter(s_acc, [idx], tsum)
            plsc.addupdate_scatter(f_acc, [idx], ones)

    def compute_row(slot):
        def px_body(jj, c2):
            off = jj * 32
            do_vreg(slot, off)
            do_vreg(slot, off + 16)
            return c2

        lax.fori_loop(0, W // 32, px_body, 0)

    start_row(0, 0)

    def pair_body(pr, carry):
        r0 = pr * 2
        start_row(r0 + 1, 1)
        wait_row(0)
        compute_row(0)

        @pl.when(r0 + 2 < ROWS_PER_W)
        def _():
            start_row(r0 + 2, 0)

        wait_row(1)
        compute_row(1)
        return carry

    lax.fori_loop(0, ROWS_PER_W // 2, pair_body, 0)

    acc_s0 = zeros
    acc_s1 = zeros
    acc_f0 = zeros
    acc_f1 = zeros
    for l in range(16):
        acc_s0 = acc_s0 + s_acc[pl.ds(l * NCLS, 16)]
        acc_s1 = acc_s1 + s_acc[pl.ds(l * NCLS + 5, 16)]
        acc_f0 = acc_f0 + f_acc[pl.ds(l * NCLS, 16)]
        acc_f1 = acc_f1 + f_acc[pl.ds(l * NCLS + 5, 16)]
    svec[pl.ds(5, 16)] = acc_s1
    svec[pl.ds(0, 16)] = acc_s0
    fvec[pl.ds(5, 16)] = acc_f1
    fvec[pl.ds(0, 16)] = acc_f0

    pltpu.sync_copy(svec, s_out.at[wid])
    pltpu.sync_copy(fvec, f_out.at[wid])


_focal_main = functools.partial(
    pl.kernel,
    out_type=[
        jax.ShapeDtypeStruct((NW, 128), jnp.float32),
        jax.ShapeDtypeStruct((NW, 128), jnp.float32),
    ],
    mesh=plsc.VectorSubcoreMesh(core_axis_name="c", subcore_axis_name="s"),
    scratch_types=[
        pltpu.VMEM((2, B * NCLS, 1, W), jnp.float32),
        pltpu.VMEM((2, B, 1, 1, W), jnp.int32),
        pltpu.VMEM((256,), jnp.float32),
        pltpu.VMEM((256,), jnp.float32),
        pltpu.VMEM((ACC_PAD,), jnp.float32),
        pltpu.VMEM((ACC_PAD,), jnp.float32),
        pltpu.VMEM((128,), jnp.float32),
        pltpu.VMEM((128,), jnp.float32),
        pltpu.SemaphoreType.DMA,
        pltpu.SemaphoreType.DMA,
    ],
    compiler_params=pltpu.CompilerParams(needs_layout_passes=False),
)(_focal_main_body)


def _combine_body(s_ref, f_ref, o_ref):
    s = jnp.sum(s_ref[...], axis=0)
    f = jnp.sum(f_ref[...], axis=0)
    cw = 1.0 / jnp.log(1.1 + f * (1.0 / float(B * HW)))
    o_ref[...] = jnp.sum(cw * s).reshape(1, 1)


def kernel(input, target):
    t = target.astype(jnp.int32)
    s_tab, f_tab = _focal_main(
        input, t, jnp.asarray(_NA2_TAB), jnp.asarray(_NB_TAB)
    )
    out = pl.pallas_call(
        _combine_body,
        out_shape=jax.ShapeDtypeStruct((1, 1), jnp.float32),
    )(s_tab, f_tab)
    return out[0, 0]


# speedup vs baseline: 1.6367x; 1.0322x over previous
import functools

import jax
import jax.numpy as jnp
import numpy as np
from jax import lax
from jax.experimental import pallas as pl
from jax.experimental.pallas import tpu as pltpu
from jax.experimental.pallas import tpu_sc as plsc

NCLS = 21
B = 4
H = 512
W = 512
HW = H * W
NW = 32
ROWS_PER_W = H // NW
ACC_PAD = 16 * NCLS + 16

LN2 = 0.6931471805599453


def _ln_tables():
    k = np.arange(256)
    mk = 1.0 + k / 256.0
    mk1 = 1.0 + (k + 1) / 256.0
    bs = (np.log(mk1) - np.log(mk)) * 256.0
    as_ = np.log(mk) - bs * mk
    nb = (-bs).astype(np.float32)
    na2 = (-as_ + 127.0 * LN2).astype(np.float32)
    return na2, nb


_NA2_TAB, _NB_TAB = _ln_tables()


def _focal_main_body(in_hbm, t_hbm, na_hbm, nb_hbm, s_out, f_out, in_v, t_v,
                     na_v, nb_v, s_acc, f_acc, svec, fvec, sem0, sem1):
    wid = lax.axis_index("s") * 2 + lax.axis_index("c")
    h0 = wid * ROWS_PER_W
    iota = lax.iota(jnp.int32, 16)
    zeros = jnp.zeros((16,), jnp.float32)
    zeros_i = jnp.zeros((16,), jnp.int32)
    ones = jnp.ones((16,), jnp.float32)
    lane_base = iota * NCLS
    sems = [sem0, sem1]

    for k in range(ACC_PAD // 16):
        s_acc[pl.ds(k * 16, 16)] = zeros
        f_acc[pl.ds(k * 16, 16)] = zeros
    for k in range(128 // 16):
        svec[pl.ds(k * 16, 16)] = zeros
        fvec[pl.ds(k * 16, 16)] = zeros

    def start_row(r, slot):
        h = h0 + r
        for b in range(B):
            pltpu.async_copy(
                in_hbm.at[b, :, pl.ds(h, 1), :],
                in_v.at[slot, pl.ds(b * NCLS, NCLS)],
                sems[slot],
            )
        pltpu.async_copy(
            t_hbm.at[:, :, pl.ds(h, 1), :], t_v.at[slot], sems[slot]
        )

    def wait_row(slot):
        for b in range(B):
            pltpu.make_async_copy(
                in_hbm.at[b, :, pl.ds(0, 1), :],
                in_v.at[slot, pl.ds(b * NCLS, NCLS)],
                sems[slot],
            ).wait()
        pltpu.make_async_copy(
            t_hbm.at[:, :, pl.ds(0, 1), :], t_v.at[slot], sems[slot]
        ).wait()

    def do_vreg(slot, off):
        wvec = off + iota
        tsum = jnp.zeros((16,), jnp.float32)
        ts = []
        for b in range(B):
            tb = t_v[slot, b, 0, 0, pl.ds(off, 16)]
            ct = tb + (b * NCLS)
            pb = plsc.load_gather(in_v.at[slot], [ct, zeros_i, wvec])
            pb = jnp.maximum(pb, 1e-5)
            i = plsc.bitcast(pb, jnp.int32)
            kk = (i >> 15) & 0xFF
            m = plsc.bitcast((i & 0x007FFFFF) | 0x3F800000, jnp.float32)
            na = plsc.load_gather(na_v, [kk])
            nb = plsc.load_gather(nb_v, [kk])
            nlnp = na + nb * m - (i >> 23).astype(jnp.float32) * LN2
            omp = 1.0 - pb
            tsum = tsum + omp * omp * nlnp
            ts.append(tb)
        for b in range(B):
            idx = lane_base + ts[b]
            plsc.addupdate_scatter(s_acc, [idx], tsum)
            plsc.addupdate_scatter(f_acc, [idx], ones)

    def compute_row(slot):
        def px_body(jj, c2):
            off = jj * 32
            do_vreg(slot, off)
            do_vreg(slot, off + 16)
            return c2

        lax.fori_loop(0, W // 32, px_body, 0)

    start_row(0, 0)
    pltpu.async_copy(na_hbm, na_v, sem1)
    pltpu.async_copy(nb_hbm, nb_v, sem1)
    pltpu.make_async_copy(na_hbm, na_v, sem1).wait()
    pltpu.make_async_copy(nb_hbm, nb_v, sem1).wait()

    def pair_body(pr, carry):
        r0 = pr * 2
        start_row(r0 + 1, 1)
        wait_row(0)
        compute_row(0)

        @pl.when(r0 + 2 < ROWS_PER_W)
        def _():
            start_row(r0 + 2, 0)

        wait_row(1)
        compute_row(1)
        return carry

    lax.fori_loop(0, ROWS_PER_W // 2, pair_body, 0)

    acc_s0 = zeros
    acc_s1 = zeros
    acc_f0 = zeros
    acc_f1 = zeros
    for l in range(16):
        acc_s0 = acc_s0 + s_acc[pl.ds(l * NCLS, 16)]
        acc_s1 = acc_s1 + s_acc[pl.ds(l * NCLS + 5, 16)]
        acc_f0 = acc_f0 + f_acc[pl.ds(l * NCLS, 16)]
        acc_f1 = acc_f1 + f_acc[pl.ds(l * NCLS + 5, 16)]
    svec[pl.ds(5, 16)] = acc_s1
    svec[pl.ds(0, 16)] = acc_s0
    fvec[pl.ds(5, 16)] = acc_f1
    fvec[pl.ds(0, 16)] = acc_f0

    pltpu.async_copy(svec, s_out.at[wid], sem0)
    pltpu.async_copy(fvec, f_out.at[wid], sem1)
    pltpu.make_async_copy(svec, s_out.at[wid], sem0).wait()
    pltpu.make_async_copy(fvec, f_out.at[wid], sem1).wait()


_focal_main = functools.partial(
    pl.kernel,
    out_type=[
        jax.ShapeDtypeStruct((NW, 128), jnp.float32),
        jax.ShapeDtypeStruct((NW, 128), jnp.float32),
    ],
    mesh=plsc.VectorSubcoreMesh(core_axis_name="c", subcore_axis_name="s"),
    scratch_types=[
        pltpu.VMEM((2, B * NCLS, 1, W), jnp.float32),
        pltpu.VMEM((2, B, 1, 1, W), jnp.int32),
        pltpu.VMEM((256,), jnp.float32),
        pltpu.VMEM((256,), jnp.float32),
        pltpu.VMEM((ACC_PAD,), jnp.float32),
        pltpu.VMEM((ACC_PAD,), jnp.float32),
        pltpu.VMEM((128,), jnp.float32),
        pltpu.VMEM((128,), jnp.float32),
        pltpu.SemaphoreType.DMA,
        pltpu.SemaphoreType.DMA,
    ],
    compiler_params=pltpu.CompilerParams(needs_layout_passes=False),
)(_focal_main_body)


def _combine_body(s_ref, f_ref, o_ref):
    s = jnp.sum(s_ref[...], axis=0)
    f = jnp.sum(f_ref[...], axis=0)
    cw = 1.0 / jnp.log(1.1 + f * (1.0 / float(B * HW)))
    o_ref[...] = jnp.sum(cw * s).reshape(1, 1)


def kernel(input, target):
    t = target.astype(jnp.int32)
    s_tab, f_tab = _focal_main(
        input, t, jnp.asarray(_NA2_TAB), jnp.asarray(_NB_TAB)
    )
    out = pl.pallas_call(
        _combine_body,
        out_shape=jax.ShapeDtypeStruct((1, 1), jnp.float32),
    )(s_tab, f_tab)
    return out[0, 0]


# speedup vs baseline: 1.7830x; 1.0894x over previous
import functools

import jax
import jax.numpy as jnp
import numpy as np
from jax import lax
from jax.experimental import pallas as pl
from jax.experimental.pallas import tpu as pltpu
from jax.experimental.pallas import tpu_sc as plsc

NCLS = 21
B = 4
H = 512
W = 512
HW = H * W
NW = 32
ROWS_PER_W = H // NW
ACC_PAD = 16 * NCLS + 16

LN2 = 0.6931471805599453


def _ln_tables():
    k = np.arange(256)
    mk = 1.0 + k / 256.0
    mk1 = 1.0 + (k + 1) / 256.0
    bs = (np.log(mk1) - np.log(mk)) * 256.0
    as_ = np.log(mk) - bs * mk
    nb = (-bs).astype(np.float32)
    na2 = (-as_ + 127.0 * LN2).astype(np.float32)
    return na2, nb


_NA2_TAB, _NB_TAB = _ln_tables()


def _focal_main_body(in_hbm, t_hbm, na_hbm, nb_hbm, s_out, f_out, in_v, t_v,
                     na_v, nb_v, s_acc, f_acc, svec, fvec, sem0, sem1):
    wid = lax.axis_index("s") * 2 + lax.axis_index("c")
    h0 = wid * ROWS_PER_W
    iota = lax.iota(jnp.int32, 16)
    zeros = jnp.zeros((16,), jnp.float32)
    zeros_i = jnp.zeros((16,), jnp.int32)
    ones = jnp.ones((16,), jnp.float32)
    lane_base = iota * NCLS
    sems = [sem0, sem1]

    for k in range(ACC_PAD // 16):
        s_acc[pl.ds(k * 16, 16)] = zeros
        f_acc[pl.ds(k * 16, 16)] = zeros
    for k in range(128 // 16):
        svec[pl.ds(k * 16, 16)] = zeros
        fvec[pl.ds(k * 16, 16)] = zeros

    def start_row(r, slot):
        h = h0 + r
        for b in range(B):
            pltpu.async_copy(
                in_hbm.at[b, :, pl.ds(h, 1), :],
                in_v.at[slot, pl.ds(b * NCLS, NCLS)],
                sems[slot],
            )
        pltpu.async_copy(
            t_hbm.at[:, :, pl.ds(h, 1), :], t_v.at[slot], sems[slot]
        )

    def wait_row(slot):
        for b in range(B):
            pltpu.make_async_copy(
                in_hbm.at[b, :, pl.ds(0, 1), :],
                in_v.at[slot, pl.ds(b * NCLS, NCLS)],
                sems[slot],
            ).wait()
        pltpu.make_async_copy(
            t_hbm.at[:, :, pl.ds(0, 1), :], t_v.at[slot], sems[slot]
        ).wait()

    def do_vreg(slot, off):
        wvec = off + iota
        tsum = jnp.zeros((16,), jnp.float32)
        ts = []
        for b in range(B):
            tb = t_v[slot, b, 0, 0, pl.ds(off, 16)]
            ct = tb + (b * NCLS)
            pb = plsc.load_gather(in_v.at[slot], [ct, zeros_i, wvec])
            pb = jnp.maximum(pb, 1e-5)
            i = plsc.bitcast(pb, jnp.int32)
            kk = (i >> 15) & 0xFF
            m = plsc.bitcast((i & 0x007FFFFF) | 0x3F800000, jnp.float32)
            na = plsc.load_gather(na_v, [kk])
            nb = plsc.load_gather(nb_v, [kk])
            nlnp = na + nb * m - (i >> 23).astype(jnp.float32) * LN2
            omp = 1.0 - pb
            tsum = tsum + omp * omp * nlnp
            ts.append(tb)
        for b in range(B):
            idx = lane_base + ts[b]
            plsc.addupdate_scatter(s_acc, [idx], tsum)
            plsc.addupdate_scatter(f_acc, [idx], ones)

    def compute_row(slot):
        @plsc.parallel_loop(0, W, step=32)
        def px_body(off):
            do_vreg(slot, off)
            do_vreg(slot, off + 16)

    start_row(0, 0)
    pltpu.async_copy(na_hbm, na_v, sem1)
    pltpu.async_copy(nb_hbm, nb_v, sem1)
    pltpu.make_async_copy(na_hbm, na_v, sem1).wait()
    pltpu.make_async_copy(nb_hbm, nb_v, sem1).wait()

    def pair_body(pr, carry):
        r0 = pr * 2
        start_row(r0 + 1, 1)
        wait_row(0)
        compute_row(0)

        @pl.when(r0 + 2 < ROWS_PER_W)
        def _():
            start_row(r0 + 2, 0)

        wait_row(1)
        compute_row(1)
        return carry

    lax.fori_loop(0, ROWS_PER_W // 2, pair_body, 0)

    acc_s0 = zeros
    acc_s1 = zeros
    acc_f0 = zeros
    acc_f1 = zeros
    for l in range(16):
        acc_s0 = acc_s0 + s_acc[pl.ds(l * NCLS, 16)]
        acc_s1 = acc_s1 + s_acc[pl.ds(l * NCLS + 5, 16)]
        acc_f0 = acc_f0 + f_acc[pl.ds(l * NCLS, 16)]
        acc_f1 = acc_f1 + f_acc[pl.ds(l * NCLS + 5, 16)]
    svec[pl.ds(5, 16)] = acc_s1
    svec[pl.ds(0, 16)] = acc_s0
    fvec[pl.ds(5, 16)] = acc_f1
    fvec[pl.ds(0, 16)] = acc_f0

    pltpu.async_copy(svec, s_out.at[wid], sem0)
    pltpu.async_copy(fvec, f_out.at[wid], sem1)
    pltpu.make_async_copy(svec, s_out.at[wid], sem0).wait()
    pltpu.make_async_copy(fvec, f_out.at[wid], sem1).wait()


_focal_main = functools.partial(
    pl.kernel,
    out_type=[
        jax.ShapeDtypeStruct((NW, 128), jnp.float32),
        jax.ShapeDtypeStruct((NW, 128), jnp.float32),
    ],
    mesh=plsc.VectorSubcoreMesh(core_axis_name="c", subcore_axis_name="s"),
    scratch_types=[
        pltpu.VMEM((2, B * NCLS, 1, W), jnp.float32),
        pltpu.VMEM((2, B, 1, 1, W), jnp.int32),
        pltpu.VMEM((256,), jnp.float32),
        pltpu.VMEM((256,), jnp.float32),
        pltpu.VMEM((ACC_PAD,), jnp.float32),
        pltpu.VMEM((ACC_PAD,), jnp.float32),
        pltpu.VMEM((128,), jnp.float32),
        pltpu.VMEM((128,), jnp.float32),
        pltpu.SemaphoreType.DMA,
        pltpu.SemaphoreType.DMA,
    ],
    compiler_params=pltpu.CompilerParams(needs_layout_passes=False),
)(_focal_main_body)


def _combine_body(s_ref, f_ref, o_ref):
    s = jnp.sum(s_ref[...], axis=0)
    f = jnp.sum(f_ref[...], axis=0)
    cw = 1.0 / jnp.log(1.1 + f * (1.0 / float(B * HW)))
    o_ref[...] = jnp.sum(cw * s).reshape(1, 1)


def kernel(input, target):
    t = target.astype(jnp.int32)
    s_tab, f_tab = _focal_main(
        input, t, jnp.asarray(_NA2_TAB), jnp.asarray(_NB_TAB)
    )
    out = pl.pallas_call(
        _combine_body,
        out_shape=jax.ShapeDtypeStruct((1, 1), jnp.float32),
    )(s_tab, f_tab)
    return out[0, 0]
